# Initial kernel scaffold; baseline (speedup 1.0000x reference)
import functools

import jax
import jax.numpy as jnp
from jax import lax
from jax.experimental import pallas as pl
from jax.experimental.pallas import tpu as pltpu
from jax.experimental.pallas import tpu_sc as plsc

N = 10000
E = 320000
D = 128
NEG_SLOPE = 0.2

NP = 10240
NPR = NP // 128
NC = 2
NS = 16
NT = NC * NS
EPT = E // NT
B = 80
NB = EPT // B
RB = 512

_SC_PARAMS = pltpu.CompilerParams(needs_layout_passes=False)
_mesh = plsc.VectorSubcoreMesh(core_axis_name="c", subcore_axis_name="s")


def _k1_body(h_ref, w_ref, emb_ref, az_ref, ae_ref, z_ref, s2_ref):
    z = jnp.dot(h_ref[...], w_ref[...], preferred_element_type=jnp.float32)
    z_ref[...] = z
    s2_ref[...] = (jnp.dot(z, az_ref[...], preferred_element_type=jnp.float32)
                   + jnp.dot(emb_ref[...], ae_ref[...],
                             preferred_element_type=jnp.float32))


def _k2_body(src_hbm, dst_hbm, s_hbm, z_hbm, num_out, den_out,
             tbl_s, tbl_d, idx_s, idx_d, w2, den_loc, rows0, zbuf,
             num_sh, den_sh):
    c = lax.axis_index("c")
    s = lax.axis_index("s")
    wid = s * NC + c
    i16 = lax.iota(jnp.int32, 16)
    zeros = jnp.zeros((16,), jnp.float32)

    def _zero_rows(r, _):
        for cc in range(8):
            zbuf[r, pl.ds(cc * 16, 16)] = zeros
        return 0
    lax.fori_loop(0, 64, _zero_rows, 0)

    def _zero_den(r, _):
        for cc in range(8):
            den_loc[r, pl.ds(cc * 16, 16)] = zeros
        return 0
    lax.fori_loop(0, NPR, _zero_den, 0)

    def _zero_num_sh(g, _):
        pltpu.sync_copy(zbuf, num_sh.at[pl.ds(s * 640 + g * 64, 64)])
        return 0
    lax.fori_loop(0, 10, _zero_num_sh, 0)
    pltpu.sync_copy(zbuf.at[pl.ds(0, 5)], den_sh.at[pl.ds(s * 5, 5)])

    pltpu.sync_copy(s_hbm.at[0], tbl_s)
    pltpu.sync_copy(s_hbm.at[1], tbl_d)
    pltpu.sync_copy(src_hbm.at[pl.ds(wid * NB, NB)], idx_s)
    pltpu.sync_copy(dst_hbm.at[pl.ds(wid * NB, NB)], idx_d)

    def _mx(k, carry):
        ms, md = carry
        ms = jnp.maximum(ms, tbl_s[pl.ds(k * 16, 16)])
        md = jnp.maximum(md, tbl_d[pl.ds(k * 16, 16)])
        return ms, md
    neg = jnp.full((16,), -3.0e38, jnp.float32)
    msv, mdv = lax.fori_loop(0, NP // 16, _mx, (neg, neg))
    m_raw = jnp.max(msv) + jnp.max(mdv)
    m_sc = jnp.maximum(m_raw, NEG_SLOPE * m_raw)
    mv = jnp.full((16,), m_sc, jnp.float32)

    plsc.subcore_barrier()

    def _score(j, _):
        for k in range(B // 16):
            si = idx_s[j, pl.ds(k * 16, 16)]
            di = idx_d[j, pl.ds(k * 16, 16)]
            a = plsc.load_gather(tbl_s, [si])
            b = plsc.load_gather(tbl_d, [di])
            x = a + b
            e = jnp.where(x >= 0.0, x, NEG_SLOPE * x)
            w = jnp.exp(e - mv)
            w2[j, pl.ds(k * 16, 16)] = w
            hi = lax.shift_right_logical(di, jnp.full((16,), 7, jnp.int32))
            lo = lax.bitwise_and(di, jnp.full((16,), 127, jnp.int32))
            plsc.addupdate_scatter(den_loc, [hi, lo], w)
        return 0
    lax.fori_loop(0, NB, _score, 0)

    def _rows(j, _):
        pltpu.sync_copy(z_hbm.at[idx_s.at[j]], rows0)

        def _scale(r, __):
            wsp = plsc.load_gather(
                w2, [jnp.full((16,), j, jnp.int32), jnp.full((16,), r, jnp.int32)])
            for cc in range(8):
                rows0[r, pl.ds(cc * 16, 16)] = rows0[r, pl.ds(cc * 16, 16)] * wsp
            return 0
        lax.fori_loop(0, B, _scale, 0)
        pltpu.sync_copy(rows0, num_sh.at[idx_d.at[j]], add=True)
        return 0
    lax.fori_loop(0, NB, _rows, 0)

    for g in range(NPR // 16):
        pltpu.sync_copy(den_loc.at[pl.ds(g * 16, 16)],
                        den_sh.at[i16 + g * 16], add=True)

    plsc.subcore_barrier()

    pltpu.sync_copy(num_sh.at[pl.ds(s * 640, 640)],
                    num_out.at[c, pl.ds(s * 640, 640)])
    pltpu.sync_copy(den_sh.at[pl.ds(s * 5, 5)],
                    den_out.at[c, pl.ds(s * 5, 5)])


_k2 = functools.partial(
    pl.kernel,
    out_type=(jax.ShapeDtypeStruct((NC, NP, D), jnp.float32),
              jax.ShapeDtypeStruct((NC, NPR, D), jnp.float32)),
    mesh=_mesh,
    compiler_params=_SC_PARAMS,
    scratch_types=[
        pltpu.VMEM((NP,), jnp.float32),
        pltpu.VMEM((NP,), jnp.float32),
        pltpu.VMEM((NB, B), jnp.int32),
        pltpu.VMEM((NB, B), jnp.int32),
        pltpu.VMEM((NB, B), jnp.float32),
        pltpu.VMEM((NPR, D), jnp.float32),
        pltpu.VMEM((B, D), jnp.float32),
        pltpu.VMEM((64, D), jnp.float32),
        pltpu.VMEM_SHARED((NP, D), jnp.float32),
        pltpu.VMEM_SHARED((NPR, D), jnp.float32),
    ],
)(_k2_body)


def _k3_body(num_ref, den_ref, out_ref):
    n = num_ref[0] + num_ref[1]
    d = den_ref[0] + den_ref[1]
    d = jnp.where(d == 0.0, 1.0, d)
    out_ref[...] = n / d


def kernel(h, edge_index, embedding, W_lin, a_fc, a_embed):
    h_p = jnp.zeros((NP, D), jnp.float32).at[:N].set(h)
    emb_p = jnp.zeros((NP, D), jnp.float32).at[:N].set(embedding)
    az = jnp.concatenate([a_fc[:D], a_fc[D:]], axis=1)
    ae = jnp.concatenate([a_embed[:D], a_embed[D:]], axis=1)
    src2 = edge_index[0].reshape(NT * NB, B)
    dst2 = edge_index[1].reshape(NT * NB, B)

    z, s2 = pl.pallas_call(
        _k1_body,
        grid=(NP // RB,),
        in_specs=[
            pl.BlockSpec((RB, D), lambda i: (i, 0)),
            pl.BlockSpec((D, D), lambda i: (0, 0)),
            pl.BlockSpec((RB, D), lambda i: (i, 0)),
            pl.BlockSpec((D, 2), lambda i: (0, 0)),
            pl.BlockSpec((D, 2), lambda i: (0, 0)),
        ],
        out_specs=[
            pl.BlockSpec((RB, D), lambda i: (i, 0)),
            pl.BlockSpec((RB, 2), lambda i: (i, 0)),
        ],
        out_shape=[
            jax.ShapeDtypeStruct((NP, D), jnp.float32),
            jax.ShapeDtypeStruct((NP, 2), jnp.float32),
        ],
    )(h_p, W_lin, emb_p, az, ae)

    s_t = s2.T

    num_parts, den_parts = _k2(src2, dst2, s_t, z)

    den_r = den_parts.reshape(NC, NP, 1)
    out_p = pl.pallas_call(
        _k3_body,
        grid=(NP // RB,),
        in_specs=[
            pl.BlockSpec((NC, RB, D), lambda i: (0, i, 0)),
            pl.BlockSpec((NC, RB, 1), lambda i: (0, i, 0)),
        ],
        out_specs=pl.BlockSpec((RB, D), lambda i: (i, 0)),
        out_shape=jax.ShapeDtypeStruct((NP, D), jnp.float32),
    )(num_parts, den_r)

    return out_p[:N]


# speedup vs baseline: 21.2795x; 21.2795x over previous
import functools

import jax
import jax.numpy as jnp
from jax import lax
from jax.experimental import pallas as pl
from jax.experimental.pallas import tpu as pltpu
from jax.experimental.pallas import tpu_sc as plsc

N = 10000
E = 320000
D = 128
NEG_SLOPE = 0.2

NP = 10240
NPR = NP // 128
NC = 1
NS = 16
NT = NC * NS
EPT = E // NT
GE = 800
UNITS = GE // 16
NBUF = 5
NBG = EPT // GE
RB = 1000

_SC_PARAMS = pltpu.CompilerParams(needs_layout_passes=False)
_mesh = plsc.VectorSubcoreMesh(core_axis_name="c", subcore_axis_name="s",
                               num_cores=NC)


def _k1_body(h_ref, w_ref, emb_ref, az_ref, ae_ref, z_ref, s2_ref):
    z = jnp.dot(h_ref[...], w_ref[...], preferred_element_type=jnp.float32,
                precision=lax.Precision.HIGHEST)
    z_ref[...] = z
    s2_ref[...] = (jnp.dot(z, az_ref[...], preferred_element_type=jnp.float32,
                           precision=lax.Precision.HIGHEST)
                   + jnp.dot(emb_ref[...], ae_ref[...],
                             preferred_element_type=jnp.float32,
                             precision=lax.Precision.HIGHEST))


def _k2_body(src_hbm, dst_hbm, ssrc_hbm, sdst_hbm, z_hbm, out_hbm,
             tbl_s, tbl_d, isf, idf, w2g, den_loc,
             rows0, rows1, rows2, rows3, rows4, denw,
             num_sh, den_sh,
             sg0, sg1, sg2, sg3, sg4, ss0, ss1, ss2, ss3, ss4, si0, si1):
    c = lax.axis_index("c")
    s = lax.axis_index("s")
    wid = s * NC + c
    i16 = lax.iota(jnp.int32, 16)
    zeros = jnp.zeros((16,), jnp.float32)
    rows = [rows0, rows1, rows2, rows3, rows4]
    sem_g = [sg0, sg1, sg2, sg3, sg4]
    sem_s = [ss0, ss1, ss2, ss3, ss4]

    def _zero_rows(r, _):
        for cc in range(8):
            rows0[r, pl.ds(cc * 16, 16)] = zeros
        return 0
    lax.fori_loop(0, 16, _zero_rows, 0)

    def _zero_den(r, _):
        for cc in range(8):
            den_loc[r, pl.ds(cc * 16, 16)] = zeros
        return 0
    lax.fori_loop(0, NPR, _zero_den, 0)

    def _zero_num_sh(g, _):
        pltpu.sync_copy(rows0, num_sh.at[pl.ds(s * 640 + g * 16, 16)])
        return 0
    lax.fori_loop(0, 40, _zero_num_sh, 0)

    @pl.when(s < 12)
    def _zden():
        pltpu.sync_copy(rows0.at[pl.ds(0, 8)], den_sh.at[pl.ds(s * 8, 8)])

    pltpu.sync_copy(ssrc_hbm, tbl_s)
    pltpu.sync_copy(sdst_hbm, tbl_d)

    def _mx(k, carry):
        ms, md = carry
        ms = jnp.maximum(ms, tbl_s[pl.ds(k * 16, 16)])
        md = jnp.maximum(md, tbl_d[pl.ds(k * 16, 16)])
        return ms, md
    neg = jnp.full((16,), -3.0e38, jnp.float32)
    msv, mdv = lax.fori_loop(0, N // 16, _mx, (neg, neg))
    m_raw = jnp.max(msv) + jnp.max(mdv)
    m_sc = jnp.maximum(m_raw, NEG_SLOPE * m_raw)
    mv = jnp.full((16,), m_sc, jnp.float32)

    plsc.subcore_barrier()

    def _group(g, _):
        ci = pltpu.async_copy(src_hbm.at[wid, g], isf, si0)
        cj = pltpu.async_copy(dst_hbm.at[wid, g], idf, si1)
        ci.wait()
        desc_g = {}
        desc_s = {}
        for u in range(NBUF - 1):
            desc_g[u] = pltpu.async_copy(
                z_hbm.at[isf[pl.ds(u * 16, 16)]], rows[u % NBUF],
                sem_g[u % NBUF])
        cj.wait()
        for u in range(UNITS):
            b = u % NBUF
            si = isf[pl.ds(u * 16, 16)]
            di = idf[pl.ds(u * 16, 16)]
            a = plsc.load_gather(tbl_s, [si])
            bb = plsc.load_gather(tbl_d, [di])
            x = a + bb
            e = jnp.where(x >= 0.0, x, NEG_SLOPE * x)
            w = jnp.exp(e - mv)
            w2g[pl.ds(u * 16, 16)] = w
            hi = lax.shift_right_logical(di, jnp.full((16,), 7, jnp.int32))
            lo = lax.bitwise_and(di, jnp.full((16,), 127, jnp.int32))
            plsc.addupdate_scatter(den_loc, [hi, lo], w)

            desc_g[u].wait()
            buf = rows[b]

            def _scale(r, __, _u=u, _buf=buf):
                wsp = plsc.load_gather(
                    w2g, [jnp.full((16,), _u * 16, jnp.int32) + r])
                for cc in range(8):
                    _buf[r, pl.ds(cc * 16, 16)] = (
                        _buf[r, pl.ds(cc * 16, 16)] * wsp)
                return 0
            lax.fori_loop(0, 16, _scale, 0)
            desc_s[u] = pltpu.async_copy(buf, num_sh.at[di], sem_s[b],
                                         add=True)
            nxt = u + NBUF - 1
            if nxt < UNITS:
                if u > 0:
                    desc_s[u - 1].wait()
                desc_g[nxt] = pltpu.async_copy(
                    z_hbm.at[isf[pl.ds(nxt * 16, 16)]], rows[nxt % NBUF],
                    sem_g[nxt % NBUF])
        for u in range(UNITS - NBUF, UNITS):
            desc_s[u].wait()
        return 0
    lax.fori_loop(0, NBG, _group, 0)

    for g in range(NPR // 16):
        pltpu.sync_copy(den_loc.at[pl.ds(g * 16, 16)],
                        den_sh.at[i16 + g * 16], add=True)

    plsc.subcore_barrier()

    base = s * 5 // 8 * 8
    pltpu.sync_copy(den_sh.at[pl.ds(base, 16)], denw)

    def _rcp(r, _):
        for cc in range(8):
            dv = denw[r, pl.ds(cc * 16, 16)]
            denw[r, pl.ds(cc * 16, 16)] = jnp.where(dv == 0.0, 0.0, 1.0 / dv)
        return 0
    lax.fori_loop(0, 16, _rcp, 0)

    def _dump(q, _):
        pltpu.sync_copy(num_sh.at[pl.ds(s * 640 + q * 16, 16)], rows0)

        def _div(r, __):
            node = jnp.full((16,), s * 640, jnp.int32) + (q * 16 + r)
            hi = lax.shift_right_logical(node, jnp.full((16,), 7, jnp.int32))
            lo = lax.bitwise_and(node, jnp.full((16,), 127, jnp.int32))
            rcp = plsc.load_gather(denw, [hi - base, lo])
            for cc in range(8):
                rows0[r, pl.ds(cc * 16, 16)] = (
                    rows0[r, pl.ds(cc * 16, 16)] * rcp)
            return 0
        lax.fori_loop(0, 16, _div, 0)
        pltpu.sync_copy(rows0, out_hbm.at[pl.ds(s * 640 + q * 16, 16)])
        return 0

    @pl.when(s < NS - 1)
    def _dump_full():
        lax.fori_loop(0, 40, _dump, 0)

    @pl.when(s == NS - 1)
    def _dump_last():
        lax.fori_loop(0, 25, _dump, 0)


_k2 = functools.partial(
    pl.kernel,
    out_type=jax.ShapeDtypeStruct((N, D), jnp.float32),
    mesh=_mesh,
    compiler_params=_SC_PARAMS,
    scratch_types=[
        pltpu.VMEM((N,), jnp.float32),
        pltpu.VMEM((N,), jnp.float32),
        pltpu.VMEM((GE,), jnp.int32),
        pltpu.VMEM((GE,), jnp.int32),
        pltpu.VMEM((GE,), jnp.float32),
        pltpu.VMEM((NPR, D), jnp.float32),
        pltpu.VMEM((16, D), jnp.float32),
        pltpu.VMEM((16, D), jnp.float32),
        pltpu.VMEM((16, D), jnp.float32),
        pltpu.VMEM((16, D), jnp.float32),
        pltpu.VMEM((16, D), jnp.float32),
        pltpu.VMEM((16, D), jnp.float32),
        pltpu.VMEM_SHARED((NP, D), jnp.float32),
        pltpu.VMEM_SHARED((96, D), jnp.float32),
        pltpu.SemaphoreType.DMA,
        pltpu.SemaphoreType.DMA,
        pltpu.SemaphoreType.DMA,
        pltpu.SemaphoreType.DMA,
        pltpu.SemaphoreType.DMA,
        pltpu.SemaphoreType.DMA,
        pltpu.SemaphoreType.DMA,
        pltpu.SemaphoreType.DMA,
        pltpu.SemaphoreType.DMA,
        pltpu.SemaphoreType.DMA,
        pltpu.SemaphoreType.DMA,
        pltpu.SemaphoreType.DMA,
    ],
)(_k2_body)


def kernel(h, edge_index, embedding, W_lin, a_fc, a_embed):
    az = jnp.concatenate([a_fc[:D], a_fc[D:]], axis=1)
    ae = jnp.concatenate([a_embed[:D], a_embed[D:]], axis=1)
    src3 = edge_index[0].reshape(NT, NBG, GE)
    dst3 = edge_index[1].reshape(NT, NBG, GE)

    z, s2 = pl.pallas_call(
        _k1_body,
        grid=(N // RB,),
        in_specs=[
            pl.BlockSpec((RB, D), lambda i: (i, 0)),
            pl.BlockSpec((D, D), lambda i: (0, 0)),
            pl.BlockSpec((RB, D), lambda i: (i, 0)),
            pl.BlockSpec((D, 2), lambda i: (0, 0)),
            pl.BlockSpec((D, 2), lambda i: (0, 0)),
        ],
        out_specs=[
            pl.BlockSpec((RB, D), lambda i: (i, 0)),
            pl.BlockSpec((RB, 2), lambda i: (i, 0)),
        ],
        out_shape=[
            jax.ShapeDtypeStruct((N, D), jnp.float32),
            jax.ShapeDtypeStruct((N, 2), jnp.float32),
        ],
    )(h, W_lin, embedding, az, ae)

    return _k2(src3, dst3, s2[:, 0], s2[:, 1], z)


# speedup vs baseline: 22.5377x; 1.0591x over previous
import functools

import jax
import jax.numpy as jnp
from jax import lax
from jax.experimental import pallas as pl
from jax.experimental.pallas import tpu as pltpu
from jax.experimental.pallas import tpu_sc as plsc

N = 10000
E = 320000
D = 128
NEG_SLOPE = 0.2

NP = 10240
NPR = NP // 128
NC = 1
NS = 16
NT = NC * NS
EPT = E // NT
GE = 800
UNITS = GE // 16
NBUF = 6
NBG = EPT // GE
RB = 1000

_SC_PARAMS = pltpu.CompilerParams(needs_layout_passes=False)
_mesh = plsc.VectorSubcoreMesh(core_axis_name="c", subcore_axis_name="s",
                               num_cores=NC)


def _k1_body(h_ref, w_ref, emb_ref, az_ref, ae_ref, z_ref, s2_ref):
    z = jnp.dot(h_ref[...], w_ref[...], preferred_element_type=jnp.float32,
                precision=lax.Precision.HIGHEST)
    z_ref[...] = z
    s2_ref[...] = (jnp.dot(z, az_ref[...], preferred_element_type=jnp.float32,
                           precision=lax.Precision.HIGHEST)
                   + jnp.dot(emb_ref[...], ae_ref[...],
                             preferred_element_type=jnp.float32,
                             precision=lax.Precision.HIGHEST))


def _k2_body(src_hbm, dst_hbm, ssrc_hbm, sdst_hbm, z_hbm, out_hbm,
             tbl_s, tbl_d, isf, idf, w2g, den_loc,
             rows0, rows1, rows2, rows3, rows4, rows5, denw,
             num_sh, den_sh,
             sg0, sg1, sg2, sg3, sg4, sg5,
             ss0, ss1, ss2, ss3, ss4, ss5, si0, si1):
    c = lax.axis_index("c")
    s = lax.axis_index("s")
    wid = s * NC + c
    i16 = lax.iota(jnp.int32, 16)
    zeros = jnp.zeros((16,), jnp.float32)
    rows = [rows0, rows1, rows2, rows3, rows4, rows5]
    sem_g = [sg0, sg1, sg2, sg3, sg4, sg5]
    sem_s = [ss0, ss1, ss2, ss3, ss4, ss5]

    def _zero_rows(r, _):
        for cc in range(8):
            rows0[r, pl.ds(cc * 16, 16)] = zeros
        return 0
    lax.fori_loop(0, 16, _zero_rows, 0)

    def _zero_den(r, _):
        for cc in range(8):
            den_loc[r, pl.ds(cc * 16, 16)] = zeros
        return 0
    lax.fori_loop(0, NPR, _zero_den, 0)

    def _zero_num_sh(g, _):
        pltpu.sync_copy(rows0, num_sh.at[pl.ds(s * 640 + g * 16, 16)])
        return 0
    lax.fori_loop(0, 40, _zero_num_sh, 0)

    @pl.when(s < 12)
    def _zden():
        pltpu.sync_copy(rows0.at[pl.ds(0, 8)], den_sh.at[pl.ds(s * 8, 8)])

    pltpu.sync_copy(ssrc_hbm, tbl_s)
    pltpu.sync_copy(sdst_hbm, tbl_d)

    def _mx(k, carry):
        ms, md = carry
        ms = jnp.maximum(ms, tbl_s[pl.ds(k * 16, 16)])
        md = jnp.maximum(md, tbl_d[pl.ds(k * 16, 16)])
        return ms, md
    neg = jnp.full((16,), -3.0e38, jnp.float32)
    msv, mdv = lax.fori_loop(0, N // 16, _mx, (neg, neg))
    m_raw = jnp.max(msv) + jnp.max(mdv)
    m_sc = jnp.maximum(m_raw, NEG_SLOPE * m_raw)
    mv = jnp.full((16,), m_sc, jnp.float32)

    plsc.subcore_barrier()

    def _group(g, _):
        ci = pltpu.async_copy(src_hbm.at[wid, g], isf, si0)
        cj = pltpu.async_copy(dst_hbm.at[wid, g], idf, si1)
        ci.wait()
        desc_g = {}
        desc_s = {}
        for u in range(NBUF - 1):
            desc_g[u] = pltpu.async_copy(
                z_hbm.at[isf[pl.ds(u * 16, 16)]], rows[u % NBUF],
                sem_g[u % NBUF])
        cj.wait()
        for u in range(UNITS):
            b = u % NBUF
            si = isf[pl.ds(u * 16, 16)]
            di = idf[pl.ds(u * 16, 16)]
            a = plsc.load_gather(tbl_s, [si])
            bb = plsc.load_gather(tbl_d, [di])
            x = a + bb
            e = jnp.where(x >= 0.0, x, NEG_SLOPE * x)
            w = jnp.exp(e - mv)
            w2g[pl.ds(u * 16, 16)] = w
            hi = lax.shift_right_logical(di, jnp.full((16,), 7, jnp.int32))
            lo = lax.bitwise_and(di, jnp.full((16,), 127, jnp.int32))
            plsc.addupdate_scatter(den_loc, [hi, lo], w)

            desc_g[u].wait()
            buf = rows[b]

            def _scale(r, __, _u=u, _buf=buf):
                wsp = plsc.load_gather(
                    w2g, [jnp.full((16,), _u * 16, jnp.int32) + r])
                for cc in range(8):
                    _buf[r, pl.ds(cc * 16, 16)] = (
                        _buf[r, pl.ds(cc * 16, 16)] * wsp)
                return 0
            lax.fori_loop(0, 16, _scale, 0)
            desc_s[u] = pltpu.async_copy(buf, num_sh.at[di], sem_s[b],
                                         add=True)
            nxt = u + NBUF - 1
            if nxt < UNITS:
                if u > 0:
                    desc_s[u - 1].wait()
                desc_g[nxt] = pltpu.async_copy(
                    z_hbm.at[isf[pl.ds(nxt * 16, 16)]], rows[nxt % NBUF],
                    sem_g[nxt % NBUF])
        for u in range(UNITS - NBUF, UNITS):
            desc_s[u].wait()
        return 0
    lax.fori_loop(0, NBG, _group, 0)

    for g in range(NPR // 16):
        pltpu.sync_copy(den_loc.at[pl.ds(g * 16, 16)],
                        den_sh.at[i16 + g * 16], add=True)

    plsc.subcore_barrier()

    base = s * 5 // 8 * 8
    pltpu.sync_copy(den_sh.at[pl.ds(base, 16)], denw)

    def _rcp(r, _):
        for cc in range(8):
            dv = denw[r, pl.ds(cc * 16, 16)]
            denw[r, pl.ds(cc * 16, 16)] = jnp.where(dv == 0.0, 0.0, 1.0 / dv)
        return 0
    lax.fori_loop(0, 16, _rcp, 0)

    def _dump(q, _):
        pltpu.sync_copy(num_sh.at[pl.ds(s * 640 + q * 16, 16)], rows0)

        def _div(r, __):
            node = jnp.full((16,), s * 640, jnp.int32) + (q * 16 + r)
            hi = lax.shift_right_logical(node, jnp.full((16,), 7, jnp.int32))
            lo = lax.bitwise_and(node, jnp.full((16,), 127, jnp.int32))
            rcp = plsc.load_gather(denw, [hi - base, lo])
            for cc in range(8):
                rows0[r, pl.ds(cc * 16, 16)] = (
                    rows0[r, pl.ds(cc * 16, 16)] * rcp)
            return 0
        lax.fori_loop(0, 16, _div, 0)
        pltpu.sync_copy(rows0, out_hbm.at[pl.ds(s * 640 + q * 16, 16)])
        return 0

    @pl.when(s < NS - 1)
    def _dump_full():
        lax.fori_loop(0, 40, _dump, 0)

    @pl.when(s == NS - 1)
    def _dump_last():
        lax.fori_loop(0, 25, _dump, 0)


_k2 = functools.partial(
    pl.kernel,
    out_type=jax.ShapeDtypeStruct((N, D), jnp.float32),
    mesh=_mesh,
    compiler_params=_SC_PARAMS,
    scratch_types=[
        pltpu.VMEM((N,), jnp.float32),
        pltpu.VMEM((N,), jnp.float32),
        pltpu.VMEM((GE,), jnp.int32),
        pltpu.VMEM((GE,), jnp.int32),
        pltpu.VMEM((GE,), jnp.float32),
        pltpu.VMEM((NPR, D), jnp.float32),
        pltpu.VMEM((16, D), jnp.float32),
        pltpu.VMEM((16, D), jnp.float32),
        pltpu.VMEM((16, D), jnp.float32),
        pltpu.VMEM((16, D), jnp.float32),
        pltpu.VMEM((16, D), jnp.float32),
        pltpu.VMEM((16, D), jnp.float32),
        pltpu.VMEM((16, D), jnp.float32),
        pltpu.VMEM_SHARED((NP, D), jnp.float32),
        pltpu.VMEM_SHARED((96, D), jnp.float32),
        pltpu.SemaphoreType.DMA,
        pltpu.SemaphoreType.DMA,
        pltpu.SemaphoreType.DMA,
        pltpu.SemaphoreType.DMA,
        pltpu.SemaphoreType.DMA,
        pltpu.SemaphoreType.DMA,
        pltpu.SemaphoreType.DMA,
        pltpu.SemaphoreType.DMA,
        pltpu.SemaphoreType.DMA,
        pltpu.SemaphoreType.DMA,
        pltpu.SemaphoreType.DMA,
        pltpu.SemaphoreType.DMA,
        pltpu.SemaphoreType.DMA,
        pltpu.SemaphoreType.DMA,
    ],
)(_k2_body)


def kernel(h, edge_index, embedding, W_lin, a_fc, a_embed):
    az = jnp.concatenate([a_fc[:D], a_fc[D:]], axis=1)
    ae = jnp.concatenate([a_embed[:D], a_embed[D:]], axis=1)
    src3 = edge_index[0].reshape(NT, NBG, GE)
    dst3 = edge_index[1].reshape(NT, NBG, GE)

    z, s2 = pl.pallas_call(
        _k1_body,
        grid=(N // RB,),
        in_specs=[
            pl.BlockSpec((RB, D), lambda i: (i, 0)),
            pl.BlockSpec((D, D), lambda i: (0, 0)),
            pl.BlockSpec((RB, D), lambda i: (i, 0)),
            pl.BlockSpec((D, 2), lambda i: (0, 0)),
            pl.BlockSpec((D, 2), lambda i: (0, 0)),
        ],
        out_specs=[
            pl.BlockSpec((RB, D), lambda i: (i, 0)),
            pl.BlockSpec((RB, 2), lambda i: (i, 0)),
        ],
        out_shape=[
            jax.ShapeDtypeStruct((N, D), jnp.float32),
            jax.ShapeDtypeStruct((N, 2), jnp.float32),
        ],
    )(h, W_lin, embedding, az, ae)

    return _k2(src3, dst3, s2[:, 0], s2[:, 1], z)
